# baseline (speedup 1.0000x reference)
import functools

import jax
import jax.numpy as jnp
from jax import lax
from jax.experimental import pallas as pl
from jax.experimental.pallas import tpu as pltpu
from jax.experimental.pallas import tpu_sc as plsc

_MAX_LEN = 8192
_DIM = 1024
_OUT_LEN = 4096
_NC = 2
_NS = 16
_L = 16
_NW = _NC * _NS
_ROWS_PER_W = _OUT_LEN // _NW
_CHUNK = 32
_NCHUNK = _ROWS_PER_W // _CHUNK

_mesh = plsc.VectorSubcoreMesh(core_axis_name="c", subcore_axis_name="s")


@functools.partial(
    pl.kernel,
    mesh=_mesh,
    out_type=jax.ShapeDtypeStruct((_OUT_LEN, _DIM), jnp.float32),
    scratch_types=[
        pltpu.VMEM((_L,), jnp.int32),
        pltpu.VMEM((_CHUNK, _DIM), jnp.float32),
        pltpu.VMEM((_CHUNK, _DIM), jnp.float32),
        pltpu.SemaphoreType.DMA,
        pltpu.SemaphoreType.DMA,
        pltpu.SemaphoreType.DMA,
        pltpu.SemaphoreType.DMA,
    ],
)
def _sc_slice_copy(len_hbm, table_hbm, out_hbm, len_v, b0, b1,
                   si0, si1, so0, so1):
    wid = lax.axis_index("s") * _NC + lax.axis_index("c")
    base = wid * _ROWS_PER_W

    pltpu.sync_copy(len_hbm, len_v)
    length = len_v[...][0]

    bufs = (b0, b1)
    sins = (si0, si1)
    souts = (so0, so1)

    ins = []
    outs = []
    for c in range(_NCHUNK):
        cb = base + c * _CHUNK
        ins.append(pltpu.make_async_copy(
            table_hbm.at[pl.ds(cb, _CHUNK)], bufs[c % 2], sins[c % 2]))
        outs.append(pltpu.make_async_copy(
            bufs[c % 2], out_hbm.at[pl.ds(cb, _CHUNK)], souts[c % 2]))

    zero = jnp.zeros((_L,), jnp.float32)

    ins[0].start()
    for c in range(_NCHUNK):
        ins[c].wait()
        if c >= 1:
            outs[c - 1].wait()
        if c + 1 < _NCHUNK:
            ins[c + 1].start()

        cb = base + c * _CHUNK
        nvalid = jnp.clip(length - cb, 0, _CHUNK)
        buf = bufs[c % 2]

        @pl.when(nvalid < _CHUNK)
        def _():
            def zero_row(r, carry):
                for j in range(_DIM // _L):
                    buf[r, pl.ds(j * _L, _L)] = zero
                return carry
            lax.fori_loop(nvalid, _CHUNK, zero_row, 0)

        outs[c].start()
    outs[_NCHUNK - 1].wait()


def kernel(length, pos_embedding):
    len_arr = jnp.broadcast_to(jnp.asarray(length, jnp.int32), (_L,))
    return _sc_slice_copy(len_arr, pos_embedding)
